# baseline (speedup 1.0000x reference)
import functools

import jax
import jax.numpy as jnp
from jax import lax
from jax.experimental import pallas as pl
from jax.experimental.pallas import tpu as pltpu
from jax.experimental.pallas import tpu_sc as plsc

VOCAB = 100000
EMBED = 64
NC = 2
NS = 16
NW = NC * NS
LANES = 16


@jax.jit
def _gather_t(idx_t, table_t):
    seq, batch = idx_t.shape

    @functools.partial(
        pl.kernel,
        out_type=jax.ShapeDtypeStruct((seq, EMBED, batch), jnp.float32),
        mesh=plsc.VectorSubcoreMesh(core_axis_name="c", subcore_axis_name="s"),
        scratch_types=[
            pltpu.VMEM((VOCAB,), jnp.float32),
            pltpu.VMEM((batch,), jnp.int32),
            pltpu.VMEM((batch,), jnp.float32),
        ],
        compiler_params=pltpu.CompilerParams(needs_layout_passes=False),
    )
    def k(idx_hbm, table_hbm, out_hbm, plane_v, xs_v, out_v):
        wid = lax.axis_index("s") * NC + lax.axis_index("c")

        for p in (0, 1):
            d = wid + NW * p
            pltpu.sync_copy(table_hbm.at[d], plane_v)

            def s_body(s, carry):
                pltpu.sync_copy(idx_hbm.at[s], xs_v)

                def c_body(c, carry2):
                    off = pl.multiple_of(c * LANES, LANES)
                    iv = xs_v[pl.ds(off, LANES)]
                    out_v[pl.ds(off, LANES)] = plsc.load_gather(plane_v, [iv])
                    return carry2

                lax.fori_loop(0, batch // LANES, c_body, 0)
                pltpu.sync_copy(out_v, out_hbm.at[s, d])
                return carry

            lax.fori_loop(0, seq, s_body, 0)

    return k(idx_t, table_t)


def kernel(x, table):
    b, s = x.shape
    out_t = _gather_t(x.T, table.T)
    return jnp.transpose(out_t, (2, 0, 1))


# speedup vs baseline: 2.2079x; 2.2079x over previous
import functools

import jax
import jax.numpy as jnp
from jax import lax
from jax.experimental import pallas as pl
from jax.experimental.pallas import tpu as pltpu
from jax.experimental.pallas import tpu_sc as plsc

VOCAB = 100000
EMBED = 64
NC = 2
NS = 16
NW = NC * NS
LANES = 16
UNROLL = 8


@jax.jit
def _gather_t(idx_t, table_t):
    seq, batch = idx_t.shape
    n_groups = batch // (LANES * UNROLL)

    @functools.partial(
        pl.kernel,
        out_type=jax.ShapeDtypeStruct((seq, EMBED, batch), jnp.float32),
        mesh=plsc.VectorSubcoreMesh(core_axis_name="c", subcore_axis_name="s"),
        scratch_types=[
            pltpu.VMEM((VOCAB,), jnp.float32),
            pltpu.VMEM((batch,), jnp.int32),
            pltpu.VMEM((batch,), jnp.int32),
            pltpu.VMEM((batch,), jnp.float32),
            pltpu.VMEM((batch,), jnp.float32),
            pltpu.SemaphoreType.DMA,
            pltpu.SemaphoreType.DMA,
            pltpu.SemaphoreType.DMA,
            pltpu.SemaphoreType.DMA,
        ],
        compiler_params=pltpu.CompilerParams(needs_layout_passes=False),
    )
    def k(idx_hbm, table_hbm, out_hbm, plane_v,
          xs0, xs1, ov0, ov1, sx0, sx1, so0, so1):
        xs = (xs0, xs1)
        ov = (ov0, ov1)
        sx = (sx0, sx1)
        so = (so0, so1)
        wid = lax.axis_index("s") * NC + lax.axis_index("c")

        def xs_wait(j):
            pltpu.make_async_copy(idx_hbm.at[0], xs[j], sx[j]).wait()

        def compute(j):
            def c_body(c, carry2):
                for u in range(UNROLL):
                    off = pl.multiple_of((c * UNROLL + u) * LANES, LANES)
                    iv = xs[j][pl.ds(off, LANES)]
                    ov[j][pl.ds(off, LANES)] = plsc.load_gather(plane_v, [iv])
                return carry2

            lax.fori_loop(0, n_groups, c_body, 0)

        for p in (0, 1):
            d = wid + NW * p
            pltpu.sync_copy(table_hbm.at[d], plane_v)
            pltpu.async_copy(idx_hbm.at[0], xs[0], sx[0])

            def s_pair(t, carry, d=d):
                for j in (0, 1):
                    s = t * 2 + j
                    if j == 0:
                        pltpu.async_copy(idx_hbm.at[s + 1], xs[1], sx[1])
                    else:
                        @pl.when(t < seq // 2 - 1)
                        def _():
                            pltpu.async_copy(idx_hbm.at[s + 1], xs[0], sx[0])
                    @pl.when(t >= 1)
                    def _():
                        pltpu.make_async_copy(
                            ov[j], out_hbm.at[0, 0], so[j]).wait()
                    xs_wait(j)
                    compute(j)
                    pltpu.async_copy(ov[j], out_hbm.at[s, d], so[j])
                return carry

            lax.fori_loop(0, seq // 2, s_pair, 0)
            pltpu.make_async_copy(ov[0], out_hbm.at[0, 0], so[0]).wait()
            pltpu.make_async_copy(ov[1], out_hbm.at[0, 0], so[1]).wait()

    return k(idx_t, table_t)


def kernel(x, table):
    b, s = x.shape
    out_t = _gather_t(x.T, table.T)
    return jnp.transpose(out_t, (2, 0, 1))


# speedup vs baseline: 2.2226x; 1.0066x over previous
import functools

import jax
import jax.numpy as jnp
from jax import lax
from jax.experimental import pallas as pl
from jax.experimental.pallas import tpu as pltpu
from jax.experimental.pallas import tpu_sc as plsc

VOCAB = 100000
EMBED = 64
NC = 2
NS = 16
NW = NC * NS
LANES = 16
UNROLL = 16


@jax.jit
def _gather_t(idx_t, table_t):
    seq, batch = idx_t.shape
    n_groups = batch // (LANES * UNROLL)

    @functools.partial(
        pl.kernel,
        out_type=jax.ShapeDtypeStruct((seq, EMBED, batch), jnp.float32),
        mesh=plsc.VectorSubcoreMesh(core_axis_name="c", subcore_axis_name="s"),
        scratch_types=[
            pltpu.VMEM((VOCAB,), jnp.float32),
            pltpu.VMEM((batch,), jnp.int32),
            pltpu.VMEM((batch,), jnp.int32),
            pltpu.VMEM((batch,), jnp.float32),
            pltpu.VMEM((batch,), jnp.float32),
            pltpu.SemaphoreType.DMA,
            pltpu.SemaphoreType.DMA,
            pltpu.SemaphoreType.DMA,
            pltpu.SemaphoreType.DMA,
        ],
        compiler_params=pltpu.CompilerParams(needs_layout_passes=False),
    )
    def k(idx_hbm, table_hbm, out_hbm, plane_v,
          xs0, xs1, ov0, ov1, sx0, sx1, so0, so1):
        xs = (xs0, xs1)
        ov = (ov0, ov1)
        sx = (sx0, sx1)
        so = (so0, so1)
        wid = lax.axis_index("s") * NC + lax.axis_index("c")

        def xs_wait(j):
            pltpu.make_async_copy(idx_hbm.at[0], xs[j], sx[j]).wait()

        def compute(j):
            def c_body(c, carry2):
                for u in range(UNROLL):
                    off = pl.multiple_of((c * UNROLL + u) * LANES, LANES)
                    iv = xs[j][pl.ds(off, LANES)]
                    ov[j][pl.ds(off, LANES)] = plsc.load_gather(plane_v, [iv])
                return carry2

            lax.fori_loop(0, n_groups, c_body, 0)

        for p in (0, 1):
            d = wid + NW * p
            pltpu.sync_copy(table_hbm.at[d], plane_v)
            pltpu.async_copy(idx_hbm.at[0], xs[0], sx[0])

            def s_pair(t, carry, d=d):
                for j in (0, 1):
                    s = t * 2 + j
                    if j == 0:
                        pltpu.async_copy(idx_hbm.at[s + 1], xs[1], sx[1])
                    else:
                        @pl.when(t < seq // 2 - 1)
                        def _():
                            pltpu.async_copy(idx_hbm.at[s + 1], xs[0], sx[0])
                    @pl.when(t >= 1)
                    def _():
                        pltpu.make_async_copy(
                            ov[j], out_hbm.at[0, 0], so[j]).wait()
                    xs_wait(j)
                    compute(j)
                    pltpu.async_copy(ov[j], out_hbm.at[s, d], so[j])
                return carry

            lax.fori_loop(0, seq // 2, s_pair, 0)
            pltpu.make_async_copy(ov[0], out_hbm.at[0, 0], so[0]).wait()
            pltpu.make_async_copy(ov[1], out_hbm.at[0, 0], so[1]).wait()

    return k(idx_t, table_t)


def kernel(x, table):
    b, s = x.shape
    out_t = _gather_t(x.T, table.T)
    return jnp.transpose(out_t, (2, 0, 1))


# speedup vs baseline: 2.2547x; 1.0144x over previous
import functools

import jax
import jax.numpy as jnp
from jax import lax
from jax.experimental import pallas as pl
from jax.experimental.pallas import tpu as pltpu
from jax.experimental.pallas import tpu_sc as plsc

VOCAB = 100000
EMBED = 64
NC = 2
NS = 16
NW = NC * NS
LANES = 16
UNROLL = 16
NBUF = 3


@jax.jit
def _gather_t(idx_t, table_t):
    seq, batch = idx_t.shape
    n_groups = batch // (LANES * UNROLL)
    n_tri = seq // NBUF

    @functools.partial(
        pl.kernel,
        out_type=jax.ShapeDtypeStruct((seq, EMBED, batch), jnp.float32),
        mesh=plsc.VectorSubcoreMesh(core_axis_name="c", subcore_axis_name="s"),
        scratch_types=[
            pltpu.VMEM((VOCAB,), jnp.float32),
            pltpu.VMEM((batch,), jnp.int32),
            pltpu.VMEM((batch,), jnp.int32),
            pltpu.VMEM((batch,), jnp.int32),
            pltpu.VMEM((batch,), jnp.float32),
            pltpu.VMEM((batch,), jnp.float32),
            pltpu.VMEM((batch,), jnp.float32),
            pltpu.SemaphoreType.DMA,
            pltpu.SemaphoreType.DMA,
            pltpu.SemaphoreType.DMA,
            pltpu.SemaphoreType.DMA,
            pltpu.SemaphoreType.DMA,
            pltpu.SemaphoreType.DMA,
        ],
        compiler_params=pltpu.CompilerParams(needs_layout_passes=False),
    )
    def k(idx_hbm, table_hbm, out_hbm, plane_v,
          xs0, xs1, xs2, ov0, ov1, ov2, sx0, sx1, sx2, so0, so1, so2):
        xs = (xs0, xs1, xs2)
        ov = (ov0, ov1, ov2)
        sx = (sx0, sx1, sx2)
        so = (so0, so1, so2)
        wid = lax.axis_index("s") * NC + lax.axis_index("c")

        def xs_wait(j):
            pltpu.make_async_copy(idx_hbm.at[0], xs[j], sx[j]).wait()

        def out_drain(j):
            pltpu.make_async_copy(ov[j], out_hbm.at[0, 0], so[j]).wait()

        def compute(j):
            def c_body(c, carry2):
                for u in range(UNROLL):
                    off = pl.multiple_of((c * UNROLL + u) * LANES, LANES)
                    iv = xs[j][pl.ds(off, LANES)]
                    ov[j][pl.ds(off, LANES)] = plsc.load_gather(plane_v, [iv])
                return carry2

            lax.fori_loop(0, n_groups, c_body, 0)

        for p in (0, 1):
            d = wid + NW * p
            pltpu.sync_copy(table_hbm.at[d], plane_v)
            pltpu.async_copy(idx_hbm.at[0], xs[0], sx[0])
            pltpu.async_copy(idx_hbm.at[1], xs[1], sx[1])

            def s_tri(t, carry, d=d):
                for j in range(NBUF):
                    s = t * NBUF + j
                    jn = (j + 2) % NBUF
                    pltpu.async_copy(idx_hbm.at[s + 2], xs[jn], sx[jn])
                    @pl.when(t >= 1)
                    def _():
                        out_drain(j)
                    xs_wait(j)
                    compute(j)
                    pltpu.async_copy(ov[j], out_hbm.at[s, d], so[j])
                return carry

            lax.fori_loop(0, n_tri, s_tri, 0)

            for s, j in ((seq - 2, 0), (seq - 1, 1)):
                out_drain(j)
                xs_wait(j)
                compute(j)
                pltpu.async_copy(ov[j], out_hbm.at[s, d], so[j])
            out_drain(2)
            out_drain(0)
            out_drain(1)

    return k(idx_t, table_t)


def kernel(x, table):
    b, s = x.shape
    out_t = _gather_t(x.T, table.T)
    return jnp.transpose(out_t, (2, 0, 1))


# speedup vs baseline: 4.0006x; 1.7744x over previous
import functools

import jax
import jax.numpy as jnp
from jax import lax
from jax.experimental import pallas as pl
from jax.experimental.pallas import tpu as pltpu
from jax.experimental.pallas import tpu_sc as plsc

VOCAB = 100000
EMBED = 64
NC = 2
NS = 16
NW = NC * NS
LANES = 16
UNROLL = 16
NBUF = 3


@jax.jit
def _gather_t(idx_t, table_t):
    seq, batch = idx_t.shape
    n_groups = batch // (LANES * UNROLL)
    n_tri = seq // NBUF

    @functools.partial(
        pl.kernel,
        out_type=jax.ShapeDtypeStruct((seq, EMBED, batch), jnp.float32),
        mesh=plsc.VectorSubcoreMesh(core_axis_name="c", subcore_axis_name="s"),
        scratch_types=[
            pltpu.VMEM((VOCAB,), jnp.float32),
            pltpu.VMEM((batch,), jnp.int32),
            pltpu.VMEM((batch,), jnp.int32),
            pltpu.VMEM((batch,), jnp.int32),
            pltpu.VMEM((batch,), jnp.float32),
            pltpu.VMEM((batch,), jnp.float32),
            pltpu.VMEM((batch,), jnp.float32),
            pltpu.SemaphoreType.DMA,
            pltpu.SemaphoreType.DMA,
            pltpu.SemaphoreType.DMA,
            pltpu.SemaphoreType.DMA,
            pltpu.SemaphoreType.DMA,
            pltpu.SemaphoreType.DMA,
        ],
        compiler_params=pltpu.CompilerParams(needs_layout_passes=False),
    )
    def k(idx_hbm, table_hbm, out_hbm, plane_v,
          xs0, xs1, xs2, ov0, ov1, ov2, sx0, sx1, sx2, so0, so1, so2):
        xs = (xs0, xs1, xs2)
        ov = (ov0, ov1, ov2)
        sx = (sx0, sx1, sx2)
        so = (so0, so1, so2)
        wid = lax.axis_index("s") * NC + lax.axis_index("c")

        def xs_wait(j):
            pltpu.make_async_copy(idx_hbm.at[0], xs[j], sx[j]).wait()

        def out_drain(j):
            pltpu.make_async_copy(ov[j], out_hbm.at[0, 0], so[j]).wait()

        def compute(j):
            @plsc.parallel_loop(0, batch // LANES, unroll=UNROLL)
            def c_body(c):
                off = pl.multiple_of(c * LANES, LANES)
                iv = xs[j][pl.ds(off, LANES)]
                ov[j][pl.ds(off, LANES)] = plsc.load_gather(plane_v, [iv])

        for p in (0, 1):
            d = wid + NW * p
            pltpu.sync_copy(table_hbm.at[d], plane_v)
            pltpu.async_copy(idx_hbm.at[0], xs[0], sx[0])
            pltpu.async_copy(idx_hbm.at[1], xs[1], sx[1])

            def s_tri(t, carry, d=d):
                for j in range(NBUF):
                    s = t * NBUF + j
                    jn = (j + 2) % NBUF
                    pltpu.async_copy(idx_hbm.at[s + 2], xs[jn], sx[jn])
                    @pl.when(t >= 1)
                    def _():
                        out_drain(j)
                    xs_wait(j)
                    compute(j)
                    pltpu.async_copy(ov[j], out_hbm.at[s, d], so[j])
                return carry

            lax.fori_loop(0, n_tri, s_tri, 0)

            for s, j in ((seq - 2, 0), (seq - 1, 1)):
                out_drain(j)
                xs_wait(j)
                compute(j)
                pltpu.async_copy(ov[j], out_hbm.at[s, d], so[j])
            out_drain(2)
            out_drain(0)
            out_drain(1)

    return k(idx_t, table_t)


def kernel(x, table):
    b, s = x.shape
    out_t = _gather_t(x.T, table.T)
    return jnp.transpose(out_t, (2, 0, 1))
